# baseline (device time: 45115 ns/iter reference)
import jax
import jax.numpy as jnp
from jax import lax
from jax.experimental import pallas as pl
from jax.experimental.pallas import tpu as pltpu

N_DEV = 4
B, SQ, D_MODEL = 2, 256, 512
H_TOT, H_LOC, DH = 16, 4, 64
SKV_SH = 256
WIN = 128
SKV1 = 128
KV_COLS = SKV_SH + SKV1

VARIANT = "full"


def kernel(x, Wq, K_ext, V_ext, Wo):
    do_comm = VARIANT in ("full", "nocompute")
    do_compute = VARIANT in ("full", "nocomm")

    def body(x_ref, wq_ref, k_ref, v_ref, wo_ref, out_ref,
             kraw, vraw, kts, vts, ktr, vtr, cbuf,
             copy_sems, kv_send_sems, kv_recv_sems,
             bf_send_sems, bf_recv_sems):
        me = lax.axis_index("i")

        if do_comm:
            @pl.when(me < 2)
            def _():
                kcp = pltpu.make_async_copy(k_ref, kraw, copy_sems.at[0])
                kcp.start()
                vcp = pltpu.make_async_copy(v_ref, vraw, copy_sems.at[1])
                vcp.start()
                kcp.wait()
                kts[...] = jnp.transpose(
                    kraw[...].astype(jnp.bfloat16), (0, 2, 3, 1))
                vcp.wait()
                vts[...] = jnp.transpose(
                    vraw[...].astype(jnp.bfloat16), (0, 2, 1, 3))

            @pl.when(me == 0)
            def _():
                ktr[:, :, :, 0:SKV_SH] = kts[:, 0:H_LOC]
                vtr[:, :, 0:SKV_SH] = vts[:, 0:H_LOC]

            @pl.when(me == 1)
            def _():
                ktr[:, :, :, SKV_SH:KV_COLS] = kts[:, H_LOC:2 * H_LOC, :, 0:SKV1]
                vtr[:, :, SKV_SH:KV_COLS] = vts[:, H_LOC:2 * H_LOC, 0:SKV1]

        xb = x_ref[...].astype(jnp.bfloat16)
        wqb = wq_ref[...].astype(jnp.bfloat16)
        wob = wo_ref[...].astype(jnp.bfloat16)
        q_all = []
        if do_compute:
            for b in range(B):
                q_all.append(
                    (jnp.dot(xb[b], wqb,
                             preferred_element_type=jnp.float32) * 0.125
                     ).astype(jnp.bfloat16))

        qi = lax.broadcasted_iota(jnp.int32, (SQ, KV_COLS), 0)
        ki = lax.broadcasted_iota(jnp.int32, (SQ, KV_COLS), 1)
        madd = jnp.where(jnp.abs(qi - ki) <= WIN,
                         jnp.float32(0), jnp.float32(-1e9))

        if do_comm or VARIANT == "barrieronly":
            barrier_sem = pltpu.get_barrier_semaphore()
            for d in range(1, N_DEV):
                pl.semaphore_signal(
                    barrier_sem, inc=1,
                    device_id=((me + d) % N_DEV,),
                    device_id_type=pl.DeviceIdType.MESH,
                )
            pl.semaphore_wait(barrier_sem, N_DEV - 1)

            @pl.when(me == 0)
            def _():
                for h in range(H_LOC):
                    for n, j in enumerate((2, 3, 1)):
                        pltpu.make_async_remote_copy(
                            src_ref=kts.at[:, 4 * j + h],
                            dst_ref=ktr.at[:, h, :, 0:SKV_SH],
                            send_sem=kv_send_sems.at[n, h],
                            recv_sem=kv_recv_sems.at[0, h],
                            device_id=(j,),
                            device_id_type=pl.DeviceIdType.MESH,
                        ).start()
                for n, j in enumerate((2, 3, 1)):
                    pltpu.make_async_remote_copy(
                        src_ref=vts.at[:, pl.ds(4 * j, H_LOC)],
                        dst_ref=vtr.at[:, :, 0:SKV_SH],
                        send_sem=kv_send_sems.at[n, H_LOC],
                        recv_sem=kv_recv_sems.at[0, H_LOC],
                        device_id=(j,),
                        device_id_type=pl.DeviceIdType.MESH,
                    ).start()

            @pl.when(me == 1)
            def _():
                for h in range(H_LOC):
                    for n, j in enumerate((3, 2, 0)):
                        pltpu.make_async_remote_copy(
                            src_ref=kts.at[:, 4 * j + h, :, 0:SKV1],
                            dst_ref=ktr.at[:, h, :, SKV_SH:KV_COLS],
                            send_sem=kv_send_sems.at[n, h],
                            recv_sem=kv_recv_sems.at[1, h],
                            device_id=(j,),
                            device_id_type=pl.DeviceIdType.MESH,
                        ).start()
                for n, j in enumerate((3, 2, 0)):
                    pltpu.make_async_remote_copy(
                        src_ref=vts.at[:, pl.ds(4 * j, H_LOC), 0:SKV1],
                        dst_ref=vtr.at[:, :, SKV_SH:KV_COLS],
                        send_sem=kv_send_sems.at[n, H_LOC],
                        recv_sem=kv_recv_sems.at[1, H_LOC],
                        device_id=(j,),
                        device_id_type=pl.DeviceIdType.MESH,
                    ).start()

        def wait_k(src, h):
            kcols = slice(0, SKV_SH) if src == 0 else slice(SKV_SH, KV_COLS)
            ssrc = slice(0, SKV_SH) if src == 0 else slice(0, SKV1)
            pltpu.make_async_remote_copy(
                src_ref=kts.at[:, h, :, ssrc],
                dst_ref=ktr.at[:, h, :, kcols],
                send_sem=kv_send_sems.at[0, h],
                recv_sem=kv_recv_sems.at[src, h],
                device_id=(src,),
                device_id_type=pl.DeviceIdType.MESH,
            ).wait_recv()

        def wait_v(src):
            kcols = slice(0, SKV_SH) if src == 0 else slice(SKV_SH, KV_COLS)
            ssrc = slice(0, SKV_SH) if src == 0 else slice(0, SKV1)
            pltpu.make_async_remote_copy(
                src_ref=vts.at[:, 0:H_LOC, ssrc],
                dst_ref=vtr.at[:, :, kcols],
                send_sem=kv_send_sems.at[0, H_LOC],
                recv_sem=kv_recv_sems.at[src, H_LOC],
                device_id=(src,),
                device_id_type=pl.DeviceIdType.MESH,
            ).wait_recv()

        p1 = me ^ 1
        p2 = 3 - me
        HALF = SQ // 2
        CHUNKS = [(b, off) for b in range(B) for off in (0, HALF)]
        ph1 = {}
        ph2 = {}
        for c, (b, off) in enumerate(CHUNKS):
            ph1[c] = pltpu.make_async_remote_copy(
                src_ref=cbuf.at[0, b, pl.ds(off, HALF)],
                dst_ref=cbuf.at[1, b, pl.ds(off, HALF)],
                send_sem=bf_send_sems.at[0, c], recv_sem=bf_recv_sems.at[0, c],
                device_id=(p1,), device_id_type=pl.DeviceIdType.MESH,
            )
            ph2[c] = pltpu.make_async_remote_copy(
                src_ref=cbuf.at[2, b, pl.ds(off, HALF)],
                dst_ref=cbuf.at[3, b, pl.ds(off, HALF)],
                send_sem=bf_send_sems.at[1, c], recv_sem=bf_recv_sems.at[1, c],
                device_id=(p2,), device_id_type=pl.DeviceIdType.MESH,
            )

        if do_comm and not do_compute:
            for src in range(2):
                @pl.when(me != src)
                def _(src=src):
                    for h in range(H_LOC):
                        wait_k(src, h)
                    wait_v(src)

        pb_all = []
        for b in range(B):
            if do_compute:
                w_b = []
                denom_b = []
                for h in range(H_LOC):
                    if do_comm and b == 0:
                        for src in range(2):
                            @pl.when(me != src)
                            def _(src=src, h=h):
                                wait_k(src, h)
                    qh = q_all[b][:, h * DH:(h + 1) * DH]
                    s = jnp.dot(qh, ktr[b, h],
                                preferred_element_type=jnp.float32) + madd
                    w = jnp.exp(s)
                    denom_b.append(jnp.sum(w, axis=1, keepdims=True))
                    w_b.append(w.astype(jnp.bfloat16))
            if do_comm and do_compute and b == 0:
                for src in range(2):
                    @pl.when(me != src)
                    def _(src=src):
                        wait_v(src)
            if do_compute:
                ctx_parts = []
                for h in range(H_LOC):
                    ctx = jnp.dot(w_b[h], vtr[b, h],
                                  preferred_element_type=jnp.float32)
                    ctx_parts.append((ctx / denom_b[h]).astype(jnp.bfloat16))
                ctx_b = jnp.concatenate(ctx_parts, axis=1)
                pb = jnp.dot(ctx_b, wob, preferred_element_type=jnp.float32)
            else:
                pb = jnp.zeros((SQ, D_MODEL), jnp.float32)
            pb_all.append(pb)
            cbuf[0, b] = pb.astype(jnp.bfloat16)
            if do_comm:
                ph1[2 * b].start()
                ph1[2 * b + 1].start()

        if do_comm:
            acc_all = {}
            for c, (b, off) in enumerate(CHUNKS):
                ph1[c].wait_recv()
                acc = (pb_all[b][off:off + HALF]
                       + cbuf[1, b, off:off + HALF].astype(jnp.float32))
                acc_all[c] = acc
                cbuf[2, b, off:off + HALF] = acc.astype(jnp.bfloat16)
                ph2[c].start()
            for c, (b, off) in enumerate(CHUNKS):
                ph2[c].wait_recv()
                out_ref[b, off:off + HALF] = (
                    acc_all[c] + cbuf[3, b, off:off + HALF].astype(jnp.float32))

            for c in range(len(CHUNKS)):
                ph1[c].wait_send()
                ph2[c].wait_send()

            def retire_sends(src):
                ssrc = slice(0, SKV_SH) if src == 0 else slice(0, SKV1)
                kcols = slice(0, SKV_SH) if src == 0 else slice(SKV_SH, KV_COLS)
                for n in range(N_DEV - 1):
                    for h in range(H_LOC):
                        pltpu.make_async_remote_copy(
                            src_ref=kts.at[:, h, :, ssrc],
                            dst_ref=ktr.at[:, h, :, kcols],
                            send_sem=kv_send_sems.at[n, h],
                            recv_sem=kv_recv_sems.at[src, h],
                            device_id=(0,),
                            device_id_type=pl.DeviceIdType.MESH,
                        ).wait_send()
                    pltpu.make_async_remote_copy(
                        src_ref=vts.at[:, 0:H_LOC, ssrc],
                        dst_ref=vtr.at[:, :, kcols],
                        send_sem=kv_send_sems.at[n, H_LOC],
                        recv_sem=kv_recv_sems.at[src, H_LOC],
                        device_id=(0,),
                        device_id_type=pl.DeviceIdType.MESH,
                    ).wait_send()

            @pl.when(me == 0)
            def _():
                retire_sends(0)

            @pl.when(me == 1)
            def _():
                retire_sends(1)
        else:
            for b in range(B):
                out_ref[b] = pb_all[b]

    return pl.pallas_call(
        body,
        out_shape=jax.ShapeDtypeStruct((B, SQ, D_MODEL), jnp.float32),
        in_specs=[
            pl.BlockSpec(memory_space=pltpu.VMEM),
            pl.BlockSpec(memory_space=pltpu.VMEM),
            pl.BlockSpec(memory_space=pltpu.MemorySpace.HBM),
            pl.BlockSpec(memory_space=pltpu.MemorySpace.HBM),
            pl.BlockSpec(memory_space=pltpu.VMEM),
        ],
        out_specs=pl.BlockSpec(memory_space=pltpu.VMEM),
        scratch_shapes=[
            pltpu.VMEM((B, SKV_SH, H_TOT, DH), jnp.float32),
            pltpu.VMEM((B, SKV_SH, H_TOT, DH), jnp.float32),
            pltpu.VMEM((B, H_TOT, DH, SKV_SH), jnp.bfloat16),
            pltpu.VMEM((B, H_TOT, SKV_SH, DH), jnp.bfloat16),
            pltpu.VMEM((B, H_LOC, DH, KV_COLS), jnp.bfloat16),
            pltpu.VMEM((B, H_LOC, KV_COLS, DH), jnp.bfloat16),
            pltpu.VMEM((4, B, SQ, D_MODEL), jnp.bfloat16),
            pltpu.SemaphoreType.DMA((2,)),
            pltpu.SemaphoreType.DMA((N_DEV - 1, H_LOC + 1)),
            pltpu.SemaphoreType.DMA((2, H_LOC + 1)),
            pltpu.SemaphoreType.DMA((2, 2 * B)),
            pltpu.SemaphoreType.DMA((2, 2 * B)),
        ],
        compiler_params=(pltpu.CompilerParams(collective_id=0)
                         if (do_comm or VARIANT == "barrieronly")
                         else pltpu.CompilerParams()),
    )(x, Wq, K_ext, V_ext, Wo)
